# baseline (speedup 1.0000x reference)
import functools

import jax
import jax.numpy as jnp
from jax import lax
from jax.experimental import pallas as pl
from jax.experimental.pallas import tpu as pltpu
from jax.experimental.pallas import tpu_sc as plsc

NUM_BIOMES = 11
EMBED_DIM = 64
BATCH = 16384
_ROWS = 12
_PAIR_D = 2 * EMBED_DIM

_info = plsc.get_sparse_core_info()
_NC, _NS = _info.num_cores, _info.num_subcores
_NW = _NC * _NS
_B_PER_W = BATCH // _NW
_P_PER_W = _B_PER_W // 2
_CHUNK = 128
_N_CHUNK = _P_PER_W // _CHUNK
_L = 16


def _make_gather():
    mesh = plsc.VectorSubcoreMesh(core_axis_name="c", subcore_axis_name="s")

    @functools.partial(
        pl.kernel,
        mesh=mesh,
        out_type=jax.ShapeDtypeStruct((BATCH // 2, _PAIR_D), jnp.float32),
        compiler_params=pltpu.CompilerParams(
            skip_device_barrier=True,
            disable_semaphore_checks=True,
        ),
        scratch_types=[
            pltpu.VMEM((_B_PER_W // _CHUNK, _CHUNK), jnp.int32),
            pltpu.VMEM((_N_CHUNK, _CHUNK), jnp.int32),
            pltpu.VMEM((_P_PER_W, _PAIR_D), jnp.float32),
            pltpu.SemaphoreType.DMA,
        ],
    )
    def gather_kernel(idx_hbm, t2_hbm, out_hbm, idx_v, pair_v, rows_v, sem):
        sid = lax.axis_index("s")
        wid = sid * _NC + lax.axis_index("c")
        base = wid * _B_PER_W
        for j in range(_B_PER_W // _CHUNK):
            pltpu.sync_copy(idx_hbm.at[pl.ds(base + j * _CHUNK, _CHUNK)],
                            idx_v.at[j])
        iota = lax.iota(jnp.int32, _L)
        lo_half = iota < (_L // 2)
        pat_e_lo = jnp.minimum(2 * iota, _L - 1)
        pat_e_hi = jnp.clip(2 * iota - _L, 0, _L - 1)
        pat_o_lo = jnp.minimum(2 * iota + 1, _L - 1)
        pat_o_hi = jnp.clip(2 * iota - _L + 1, 0, _L - 1)
        for g in range(_B_PER_W // (2 * _L)):
            flat = g * 2 * _L
            a = idx_v[flat // _CHUNK, pl.ds(flat % _CHUNK, _L)]
            b = idx_v[(flat + _L) // _CHUNK, pl.ds((flat + _L) % _CHUNK, _L)]
            ev = jnp.where(lo_half, jnp.take(a, pat_e_lo), jnp.take(b, pat_e_hi))
            od = jnp.where(lo_half, jnp.take(a, pat_o_lo), jnp.take(b, pat_o_hi))
            pair = ev * _ROWS + od
            pair_v[g * _L // _CHUNK, pl.ds((g * _L) % _CHUNK, _L)] = pair
        copies = []
        for j in range(_N_CHUNK):
            copies.append(pltpu.async_copy(
                t2_hbm.at[pair_v.at[j]],
                rows_v.at[pl.ds(j * _CHUNK, _CHUNK)],
                sem,
            ))
        for c in copies:
            c.wait()
        pltpu.sync_copy(rows_v, out_hbm.at[pl.ds(wid * _P_PER_W, _P_PER_W)])

    return gather_kernel


_gather = _make_gather()


def kernel(biome_labels, table):
    idx = biome_labels.astype(jnp.int32)
    padded = jnp.pad(table, ((0, _ROWS - NUM_BIOMES), (0, 0)))
    t2 = jnp.concatenate(
        [jnp.repeat(padded, _ROWS, axis=0), jnp.tile(padded, (_ROWS, 1))],
        axis=1,
    )
    paired = _gather(idx, t2)
    return paired.reshape(BATCH, EMBED_DIM)
